# baseline (device time: 108561 ns/iter reference)
import jax
import jax.numpy as jnp
from jax import lax
from jax.experimental import pallas as pl
from jax.experimental.pallas import tpu as pltpu

N_DEV = 32


def kernel(x, router_W, route_idx, expert_W, shared_W):
    n_tok, d_model = x.shape
    n_exp = router_W.shape[1]
    n_exp_per, _, d_ff = expert_W.shape

    x_bf = x.astype(jnp.bfloat16)
    rw_bf = router_W.astype(jnp.bfloat16)
    ew_bf = expert_W.astype(jnp.bfloat16)
    sw_bf = shared_W.astype(jnp.bfloat16)

    def body(x_ref, rw_ref, idx_ref, ew_ref, sw_ref, out_ref,
             comm_ref, send_sems, recv_sems):
        my = lax.axis_index("i")
        left = (my - 1) % N_DEV
        right = (my + 1) % N_DEV

        barrier_sem = pltpu.get_barrier_semaphore()
        pl.semaphore_signal(barrier_sem, inc=1, device_id=(left,),
                            device_id_type=pl.DeviceIdType.MESH)
        pl.semaphore_signal(barrier_sem, inc=1, device_id=(right,),
                            device_id_type=pl.DeviceIdType.MESH)
        pl.semaphore_wait(barrier_sem, 2)

        xv = x_ref[...]
        idx = idx_ref[...]

        def make_rdma(h):
            src = ew_ref if h == 0 else comm_ref.at[h - 1]
            return pltpu.make_async_remote_copy(
                src_ref=src,
                dst_ref=comm_ref.at[h],
                send_sem=send_sems.at[h],
                recv_sem=recv_sems.at[h],
                device_id=(right,),
                device_id_type=pl.DeviceIdType.MESH,
            )

        rdma = make_rdma(0)
        rdma.start()

        scores = jnp.dot(xv, rw_ref[...],
                         preferred_element_type=jnp.float32)
        s_max = jnp.max(scores, axis=1, keepdims=True)
        p = jnp.exp(scores - s_max)
        probs = p / jnp.sum(p, axis=1, keepdims=True)
        e_iota = lax.broadcasted_iota(jnp.int32, (n_tok, n_exp), 1)
        gate = jnp.sum(jnp.where(e_iota == idx, probs, 0.0),
                       axis=1, keepdims=True)

        out_ref[...] = jnp.dot(xv, sw_ref[...],
                               preferred_element_type=jnp.float32)

        def accum(load, origin):
            for j in range(n_exp_per):
                e = origin * n_exp_per + j
                coeff = jnp.where(idx == e, gate, 0.0)
                y = jnp.dot(xv, load(j),
                            preferred_element_type=jnp.float32)
                out_ref[...] += coeff * y

        accum(lambda j: ew_ref[j], my)
        rdma.wait()

        for h in range(1, N_DEV - 1):
            rdma = make_rdma(h)
            rdma.start()
            accum(lambda j, h=h: comm_ref[h - 1, j], (my - h) % N_DEV)
            rdma.wait()

        accum(lambda j: comm_ref[N_DEV - 2, j], (my + 1) % N_DEV)

    return pl.pallas_call(
        body,
        out_shape=jax.ShapeDtypeStruct((n_tok, d_ff), jnp.float32),
        in_specs=[pl.BlockSpec(memory_space=pltpu.VMEM)] * 5,
        out_specs=pl.BlockSpec(memory_space=pltpu.VMEM),
        scratch_shapes=[
            pltpu.VMEM((N_DEV - 1, n_exp_per, d_model, d_ff), jnp.bfloat16),
            pltpu.SemaphoreType.DMA((N_DEV - 1,)),
            pltpu.SemaphoreType.DMA((N_DEV - 1,)),
        ],
        compiler_params=pltpu.CompilerParams(collective_id=0),
    )(x_bf, rw_bf, route_idx, ew_bf, sw_bf)


# device time: 70463 ns/iter; 1.5407x vs baseline; 1.5407x over previous
import jax
import jax.numpy as jnp
from jax import lax
from jax.experimental import pallas as pl
from jax.experimental.pallas import tpu as pltpu

N_DEV = 32


def kernel(x, router_W, route_idx, expert_W, shared_W):
    n_tok, d_model = x.shape
    n_exp = router_W.shape[1]
    n_exp_per, _, d_ff = expert_W.shape

    x_bf = x.astype(jnp.bfloat16)
    rw_bf = router_W.astype(jnp.bfloat16)
    ew_bf = expert_W.astype(jnp.bfloat16)
    sw_bf = shared_W.astype(jnp.bfloat16)

    def body(x_ref, rw_ref, idx_ref, ew_ref, sw_ref, out_ref,
             comm_ref, send_sems, recv_sems):
        my = lax.axis_index("i")

        barrier_sem = pltpu.get_barrier_semaphore()
        for k in range(1, N_DEV):
            pl.semaphore_signal(barrier_sem, inc=1,
                                device_id=((my + k) % N_DEV,),
                                device_id_type=pl.DeviceIdType.MESH)
        pl.semaphore_wait(barrier_sem, N_DEV - 1)

        sends = []
        for k in range(1, N_DEV):
            rdma = pltpu.make_async_remote_copy(
                src_ref=ew_ref,
                dst_ref=comm_ref.at[my],
                send_sem=send_sems.at[k - 1],
                recv_sem=recv_sems.at[my],
                device_id=((my + k) % N_DEV,),
                device_id_type=pl.DeviceIdType.MESH,
            )
            rdma.start()
            sends.append(rdma)

        xv = x_ref[...]
        idx = idx_ref[...]

        scores = jnp.dot(xv, rw_ref[...],
                         preferred_element_type=jnp.float32)
        s_max = jnp.max(scores, axis=1, keepdims=True)
        p = jnp.exp(scores - s_max)
        probs = p / jnp.sum(p, axis=1, keepdims=True)
        e_iota = lax.broadcasted_iota(jnp.int32, (n_tok, n_exp), 1)
        gate = jnp.sum(jnp.where(e_iota == idx, probs, 0.0),
                       axis=1, keepdims=True)

        out_ref[...] = jnp.dot(xv, sw_ref[...],
                               preferred_element_type=jnp.float32)

        def accum(chunk, origin):
            acc = out_ref[...]
            for j in range(n_exp_per):
                e = origin * n_exp_per + j
                coeff = jnp.where(idx == e, gate, 0.0)
                y = jnp.dot(xv, chunk[j],
                            preferred_element_type=jnp.float32)
                acc += coeff * y
            out_ref[...] = acc

        accum(ew_ref[...], my)

        for k in range(1, N_DEV):
            o = (my + k) % N_DEV
            recv = pltpu.make_async_remote_copy(
                src_ref=ew_ref,
                dst_ref=comm_ref.at[o],
                send_sem=send_sems.at[0],
                recv_sem=recv_sems.at[o],
                device_id=(my,),
                device_id_type=pl.DeviceIdType.MESH,
            )
            recv.wait_recv()
            accum(comm_ref[o], o)

        for rdma in sends:
            rdma.wait_send()

    return pl.pallas_call(
        body,
        out_shape=jax.ShapeDtypeStruct((n_tok, d_ff), jnp.float32),
        in_specs=[pl.BlockSpec(memory_space=pltpu.VMEM)] * 5,
        out_specs=pl.BlockSpec(memory_space=pltpu.VMEM),
        scratch_shapes=[
            pltpu.VMEM((N_DEV, n_exp_per, d_model, d_ff), jnp.bfloat16),
            pltpu.SemaphoreType.DMA((N_DEV - 1,)),
            pltpu.SemaphoreType.DMA((N_DEV,)),
        ],
        compiler_params=pltpu.CompilerParams(collective_id=0),
    )(x_bf, rw_bf, route_idx, ew_bf, sw_bf)


# device time: 63088 ns/iter; 1.7208x vs baseline; 1.1169x over previous
import jax
import jax.numpy as jnp
from jax import lax
from jax.experimental import pallas as pl
from jax.experimental.pallas import tpu as pltpu

N_DEV = 32
N_Z = 4
N_Q = 8


def kernel(x, router_W, route_idx, expert_W, shared_W):
    n_tok, d_model = x.shape
    n_exp = router_W.shape[1]
    n_exp_per, _, d_ff = expert_W.shape
    w_cols = n_exp_per * d_ff

    x_bf = x.astype(jnp.bfloat16)
    rw_bf = router_W.astype(jnp.bfloat16)
    sw_bf = shared_W.astype(jnp.bfloat16)
    ew_q = expert_W.transpose(1, 0, 2).reshape(d_model, w_cols)
    ew_q = ew_q.astype(jnp.bfloat16)

    def body(x_ref, rw_ref, idx_ref, ew_ref, sw_ref, out_ref,
             zcomm_ref, pcomm_ref,
             z_send_sems, z_recv_sems, p_send_sems, p_recv_sems):
        my = lax.axis_index("i")
        myz = my // N_Q
        myq = my % N_Q

        barrier_sem = pltpu.get_barrier_semaphore()
        for zoff in range(1, N_Z):
            peer = ((myz + zoff) % N_Z) * N_Q + myq
            pl.semaphore_signal(barrier_sem, inc=1, device_id=(peer,),
                                device_id_type=pl.DeviceIdType.MESH)
        for qoff in range(1, N_Q):
            peer = myz * N_Q + (myq + qoff) % N_Q
            pl.semaphore_signal(barrier_sem, inc=1, device_id=(peer,),
                                device_id_type=pl.DeviceIdType.MESH)
        pl.semaphore_wait(barrier_sem, (N_Z - 1) + (N_Q - 1))

        zcomm_ref[myz] = ew_ref[...]

        z_sends = []
        for zoff in range(1, N_Z):
            peer = ((myz + zoff) % N_Z) * N_Q + myq
            rdma = pltpu.make_async_remote_copy(
                src_ref=ew_ref,
                dst_ref=zcomm_ref.at[myz],
                send_sem=z_send_sems.at[zoff - 1],
                recv_sem=z_recv_sems.at[myz],
                device_id=(peer,),
                device_id_type=pl.DeviceIdType.MESH,
            )
            rdma.start()
            z_sends.append(rdma)

        xv = x_ref[...]
        idx = idx_ref[...]

        scores = jnp.dot(xv, rw_ref[...],
                         preferred_element_type=jnp.float32)
        s_max = jnp.max(scores, axis=1, keepdims=True)
        p = jnp.exp(scores - s_max)
        probs = p / jnp.sum(p, axis=1, keepdims=True)
        e_iota = lax.broadcasted_iota(jnp.int32, (n_tok, n_exp), 1)
        gate = jnp.sum(jnp.where(e_iota == idx, probs, 0.0),
                       axis=1, keepdims=True)

        def accum(acc, chunk_q, origin):
            w = chunk_q.astype(jnp.bfloat16)
            y = jnp.dot(xv, w, preferred_element_type=jnp.float32)
            for j in range(n_exp_per):
                e = origin * n_exp_per + j
                coeff = jnp.where(idx == e, gate, 0.0)
                acc += coeff * y[:, j * d_ff:(j + 1) * d_ff]
            return acc

        acc = jnp.dot(xv, sw_ref[...], preferred_element_type=jnp.float32)
        acc = accum(acc, ew_ref[...], my)

        for zoff in range(1, N_Z):
            zsrc = (myz + zoff) % N_Z
            recv = pltpu.make_async_remote_copy(
                src_ref=ew_ref,
                dst_ref=zcomm_ref.at[zsrc],
                send_sem=z_send_sems.at[0],
                recv_sem=z_recv_sems.at[zsrc],
                device_id=(my,),
                device_id_type=pl.DeviceIdType.MESH,
            )
            recv.wait_recv()
            acc = accum(acc, zcomm_ref[zsrc], zsrc * N_Q + myq)

        p_sends = []
        for qoff in range(1, N_Q):
            peer = myz * N_Q + (myq + qoff) % N_Q
            rdma = pltpu.make_async_remote_copy(
                src_ref=zcomm_ref,
                dst_ref=pcomm_ref.at[myq],
                send_sem=p_send_sems.at[qoff - 1],
                recv_sem=p_recv_sems.at[myq],
                device_id=(peer,),
                device_id_type=pl.DeviceIdType.MESH,
            )
            rdma.start()
            p_sends.append(rdma)

        for qoff in range(1, N_Q):
            qsrc = (myq + qoff) % N_Q
            recv = pltpu.make_async_remote_copy(
                src_ref=zcomm_ref,
                dst_ref=pcomm_ref.at[qsrc],
                send_sem=p_send_sems.at[0],
                recv_sem=p_recv_sems.at[qsrc],
                device_id=(my,),
                device_id_type=pl.DeviceIdType.MESH,
            )
            recv.wait_recv()
            for z in range(N_Z):
                acc = accum(acc, pcomm_ref[qsrc, z], z * N_Q + qsrc)

        out_ref[...] = acc

        for rdma in z_sends + p_sends:
            rdma.wait_send()

    return pl.pallas_call(
        body,
        out_shape=jax.ShapeDtypeStruct((n_tok, d_ff), jnp.float32),
        in_specs=[pl.BlockSpec(memory_space=pltpu.VMEM)] * 5,
        out_specs=pl.BlockSpec(memory_space=pltpu.VMEM),
        scratch_shapes=[
            pltpu.VMEM((N_Z, d_model, w_cols), jnp.bfloat16),
            pltpu.VMEM((N_Q, N_Z, d_model, w_cols), jnp.bfloat16),
            pltpu.SemaphoreType.DMA((N_Z - 1,)),
            pltpu.SemaphoreType.DMA((N_Z,)),
            pltpu.SemaphoreType.DMA((N_Q - 1,)),
            pltpu.SemaphoreType.DMA((N_Q,)),
        ],
        compiler_params=pltpu.CompilerParams(collective_id=0),
    )(x_bf, rw_bf, route_idx, ew_q, sw_bf)
